# Initial kernel scaffold; baseline (speedup 1.0000x reference)
import jax
import jax.numpy as jnp
from jax.experimental import pallas as pl


def kernel(hidden_states, Wq, bq, Wk, bk, Wv, bv, Wproj, log_bandwidth):
    raise NotImplementedError("write your pallas kernel here")


# speedup vs baseline: 3.3411x; 3.3411x over previous
import functools

import jax
import jax.numpy as jnp
from jax.experimental import pallas as pl
from jax.experimental.pallas import tpu as pltpu

B, T, HID, H = 2, 4096, 1024, 16
D = HID // H
BH = B * H
EPS = 1e-4


def _matmul_bias_kernel(x_ref, w_ref, b_ref, o_ref):
    o_ref[...] = (
        jnp.dot(x_ref[...], w_ref[...], preferred_element_type=jnp.float32)
        + b_ref[...]
    )


def _matmul_bias(x, w, bias, bm, bn):
    m, k = x.shape
    n = w.shape[1]
    return pl.pallas_call(
        _matmul_bias_kernel,
        grid=(m // bm, n // bn),
        in_specs=[
            pl.BlockSpec((bm, k), lambda i, j: (i, 0)),
            pl.BlockSpec((k, bn), lambda i, j: (0, j)),
            pl.BlockSpec((1, bn), lambda i, j: (0, j)),
        ],
        out_specs=pl.BlockSpec((bm, bn), lambda i, j: (i, j)),
        out_shape=jax.ShapeDtypeStruct((m, n), jnp.float32),
    )(x, w, bias.reshape(1, n))


def _bump_kernel(zq_ref, zk_ref, v_ref, lbw_ref, o_ref):
    kstep = pl.program_id(2)

    @pl.when(kstep == 0)
    def _init():
        o_ref[...] = jnp.zeros_like(o_ref)

    lbw = lbw_ref[0, 0, 0]
    bw = jnp.maximum(jnp.log1p(jnp.exp(-jnp.abs(lbw))) + jnp.maximum(lbw, 0.0)
                     + EPS, EPS)
    inv_bw = 1.0 / bw
    zq = zq_ref[0, 0, :]
    zk = zk_ref[0, 0, :]
    w = jnp.maximum(1.0 - jnp.abs(zq[:, None] - zk[None, :]) * inv_bw, 0.0)
    acc = jnp.dot(w.astype(jnp.bfloat16), v_ref[0].astype(jnp.bfloat16),
                  preferred_element_type=jnp.float32)
    o_ref[...] += (acc * (1.0 / T))[None]


def _bump_attention(zq, zk, v, log_bw_bh, tq=256, tk=1024):
    return pl.pallas_call(
        _bump_kernel,
        grid=(BH, T // tq, T // tk),
        in_specs=[
            pl.BlockSpec((1, 1, tq), lambda b, q, k: (b, 0, q)),
            pl.BlockSpec((1, 1, tk), lambda b, q, k: (b, 0, k)),
            pl.BlockSpec((1, tk, D), lambda b, q, k: (b, k, 0)),
            pl.BlockSpec((1, 1, 1), lambda b, q, k: (b, 0, 0)),
        ],
        out_specs=pl.BlockSpec((1, tq, D), lambda b, q, k: (b, q, 0)),
        out_shape=jax.ShapeDtypeStruct((BH, T, D), jnp.float32),
        compiler_params=pltpu.CompilerParams(
            dimension_semantics=("parallel", "parallel", "arbitrary"),
        ),
    )(zq.reshape(BH, 1, T), zk.reshape(BH, 1, T), v,
      log_bw_bh.reshape(BH, 1, 1))


@jax.jit
def kernel(hidden_states, Wq, bq, Wk, bk, Wv, bv, Wproj, log_bandwidth):
    x = hidden_states.reshape(B * T, HID)
    w_qkv = jnp.concatenate([Wq.T, Wk.T, Wv.T], axis=1)
    b_qkv = jnp.concatenate([bq, bk, bv], axis=0)
    qkv = _matmul_bias(x, w_qkv, b_qkv, bm=512, bn=1024)
    q, k, v = jnp.split(qkv.reshape(B, T, 3 * HID), 3, axis=2)

    q_flat = q.reshape(B, T, H, D).transpose(0, 2, 1, 3).reshape(B * T, HID)
    k_flat = k.reshape(B, T, H, D).transpose(0, 2, 1, 3).reshape(B * T, HID)
    qk_flat = jnp.concatenate([q_flat, k_flat], axis=0)
    wp = jnp.zeros((HID, 128), jnp.float32).at[:, :H].set(Wproj.T)
    zqk = _matmul_bias(qk_flat, wp, jnp.zeros((128,), jnp.float32),
                       bm=1024, bn=128)[:, :H]
    zq = zqk[: B * T].reshape(B, T, H).transpose(0, 2, 1).reshape(BH, T)
    zk = zqk[B * T:].reshape(B, T, H).transpose(0, 2, 1).reshape(BH, T)

    v_h = v.reshape(B, T, H, D).transpose(0, 2, 1, 3).reshape(BH, T, D)
    log_bw_bh = jnp.broadcast_to(log_bandwidth[None, :], (B, H)).reshape(BH, 1)

    ctx = _bump_attention(zq, zk, v_h, log_bw_bh)
    return ctx.reshape(B, H, T, D).transpose(0, 2, 1, 3).reshape(B, T, HID)


# speedup vs baseline: 6.8155x; 2.0399x over previous
import functools

import jax
import jax.numpy as jnp
from jax import lax
from jax.experimental import pallas as pl
from jax.experimental.pallas import tpu as pltpu
from jax.experimental.pallas import tpu_sc as plsc

B, T, HID, H = 2, 4096, 1024, 16
D = HID // H
BH = B * H
EPS = 1e-4
L = 16
NC, NS = 2, 16
CHUNK = 128
QB = 128
TROWS = T + 8


def _matmul_bias_kernel(x_ref, w_ref, b_ref, o_ref):
    o_ref[...] = (
        jnp.dot(x_ref[...], w_ref[...], preferred_element_type=jnp.float32)
        + b_ref[...]
    )


def _matmul_bias(x, w, bias, bm, bn):
    m, k = x.shape
    n = w.shape[1]
    return pl.pallas_call(
        _matmul_bias_kernel,
        grid=(m // bm, n // bn),
        in_specs=[
            pl.BlockSpec((bm, k), lambda i, j: (i, 0)),
            pl.BlockSpec((k, bn), lambda i, j: (0, j)),
            pl.BlockSpec((1, bn), lambda i, j: (0, j)),
        ],
        out_specs=pl.BlockSpec((bm, bn), lambda i, j: (i, j)),
        out_shape=jax.ShapeDtypeStruct((m, n), jnp.float32),
    )(x, w, bias.reshape(1, n))


def _bw_kernel(l_ref, o_ref):
    x = l_ref[...]
    sp = jnp.log1p(jnp.exp(-jnp.abs(x))) + jnp.maximum(x, 0.0) + EPS
    o_ref[...] = jnp.maximum(sp, EPS)


def _bandwidths(log_bw_bh):
    return pl.pallas_call(
        _bw_kernel,
        out_shape=jax.ShapeDtypeStruct((1, BH), jnp.float32),
    )(log_bw_bh.reshape(1, BH))


def _sc_bump_body(zq_hbm, zk_hbm, v_hbm, bw_hbm, ctx_hbm, pv_hbm, pzv_hbm,
                  zk_s, zq_s, pidx, va, pvb, zvb, g0, g1, g2, g3, g4, g5,
                  outb, il, ia, ir, bwv, sem):
    w = lax.axis_index("s") * NC + lax.axis_index("c")
    base = w * T
    trow = w * TROWS
    pltpu.sync_copy(zk_hbm.at[pl.ds(base, T)], zk_s)
    pltpu.sync_copy(zq_hbm.at[pl.ds(base, T)], zq_s)
    pltpu.sync_copy(bw_hbm, bwv)
    bw16 = bwv[pl.ds(w * L, L)]
    inv_bw = 1.0 / bw16
    lane = lax.iota(jnp.int32, L)

    def _init(i, c):
        pidx[pl.ds(i * L, L)] = base + i * L + lane
        return c

    lax.fori_loop(0, T // L, _init, 0)

    def _vsort0(i, c):
        k, p = plsc.sort_key_val(zk_s[pl.ds(i * L, L)],
                                 pidx[pl.ds(i * L, L)])
        zk_s[pl.ds(i * L, L)] = k
        pidx[pl.ds(i * L, L)] = p
        return c

    lax.fori_loop(0, T // L, _vsort0, 0)

    npairs = T // (2 * L)
    for lvl in range(8):
        m = 32 << lvl
        half = max(m // (2 * L), 1)

        def _rev_merge(p, c, m=m, half=half):
            r = p // half
            i = p % half
            a0 = r * m + i * L
            b0 = r * m + m - L - i * L
            ka = zk_s[pl.ds(a0, L)]
            kb = lax.rev(zk_s[pl.ds(b0, L)], (0,))
            pa = pidx[pl.ds(a0, L)]
            pb = lax.rev(pidx[pl.ds(b0, L)], (0,))
            msk = ka <= kb
            zk_s[pl.ds(a0, L)] = jnp.where(msk, ka, kb)
            pidx[pl.ds(a0, L)] = jnp.where(msk, pa, pb)
            zk_s[pl.ds(b0, L)] = lax.rev(jnp.where(msk, kb, ka), (0,))
            pidx[pl.ds(b0, L)] = lax.rev(jnp.where(msk, pb, pa), (0,))
            return c

        lax.fori_loop(0, npairs, _rev_merge, 0)
        strd = m // 4
        while strd >= L:
            spb = strd // L

            def _cx(p, c, spb=spb, strd=strd):
                blk = p // spb
                off = p % spb
                a0 = (blk * 2 * spb + off) * L
                b0 = a0 + strd
                ka = zk_s[pl.ds(a0, L)]
                kb = zk_s[pl.ds(b0, L)]
                pa = pidx[pl.ds(a0, L)]
                pb = pidx[pl.ds(b0, L)]
                msk = ka <= kb
                zk_s[pl.ds(a0, L)] = jnp.where(msk, ka, kb)
                zk_s[pl.ds(b0, L)] = jnp.where(msk, kb, ka)
                pidx[pl.ds(a0, L)] = jnp.where(msk, pa, pb)
                pidx[pl.ds(b0, L)] = jnp.where(msk, pb, pa)
                return c

            lax.fori_loop(0, npairs, _cx, 0)
            strd //= 2

        def _vsort(i, c):
            k, p = plsc.sort_key_val(zk_s[pl.ds(i * L, L)],
                                     pidx[pl.ds(i * L, L)])
            zk_s[pl.ds(i * L, L)] = k
            pidx[pl.ds(i * L, L)] = p
            return c

        lax.fori_loop(0, T // L, _vsort, 0)

    zero = jnp.zeros((L,), jnp.float32)
    carry0 = (zero,) * 8
    for ci in range(T // CHUNK):
        pltpu.async_copy(v_hbm.at[pidx.at[pl.ds(ci * CHUNK, CHUNK)]], va,
                         sem).wait()

        def _row(i, cr, ci=ci):
            pv0, pv1, pv2, pv3, pz0, pz1, pz2, pz3 = cr
            zv = plsc.load_gather(zk_s, [jnp.full((L,), ci * CHUNK + i,
                                                  jnp.int32)])
            v0 = va[i, pl.ds(0, L)]
            v1 = va[i, pl.ds(L, L)]
            v2 = va[i, pl.ds(2 * L, L)]
            v3 = va[i, pl.ds(3 * L, L)]
            pvb[i, pl.ds(0, L)] = pv0
            pvb[i, pl.ds(L, L)] = pv1
            pvb[i, pl.ds(2 * L, L)] = pv2
            pvb[i, pl.ds(3 * L, L)] = pv3
            zvb[i, pl.ds(0, L)] = pz0
            zvb[i, pl.ds(L, L)] = pz1
            zvb[i, pl.ds(2 * L, L)] = pz2
            zvb[i, pl.ds(3 * L, L)] = pz3
            return (pv0 + v0, pv1 + v1, pv2 + v2, pv3 + v3,
                    pz0 + v0 * zv, pz1 + v1 * zv, pz2 + v2 * zv,
                    pz3 + v3 * zv)

        carry0 = lax.fori_loop(0, CHUNK, _row, carry0)
        pltpu.sync_copy(pvb, pv_hbm.at[pl.ds(trow + ci * CHUNK, CHUNK)])
        pltpu.sync_copy(zvb, pzv_hbm.at[pl.ds(trow + ci * CHUNK, CHUNK)])

    for g in range(4):
        pvb[0, pl.ds(g * L, L)] = carry0[g]
        zvb[0, pl.ds(g * L, L)] = carry0[4 + g]
    pltpu.sync_copy(pvb.at[0], pv_hbm.at[trow + T])
    pltpu.sync_copy(zvb.at[0], pzv_hbm.at[trow + T])

    one = jnp.float32(1.0)
    scale = jnp.float32(1.0 / T)

    def _batch(bi, c):
        def _ranks(qi, c2):
            q0 = bi * QB + qi * L
            zqv = zq_s[pl.ds(q0, L)]
            tl = zqv - bw16
            tm = zqv
            th = zqv + bw16
            posl = jnp.zeros((L,), jnp.int32)
            posm = jnp.zeros((L,), jnp.int32)
            posh = jnp.zeros((L,), jnp.int32)
            for k in range(12, -1, -1):
                bit = 1 << k
                for which in range(3):
                    pos = (posl, posm, posh)[which]
                    tgt = (tl, tm, th)[which]
                    npos = pos + bit
                    idx = jnp.minimum(npos, T) - 1
                    val = plsc.load_gather(zk_s, [idx])
                    pos = jnp.where((npos <= T) & (val < tgt), npos, pos)
                    if which == 0:
                        posl = pos
                    elif which == 1:
                        posm = pos
                    else:
                        posh = pos
            il[pl.ds(qi * L, L)] = posl + trow
            ia[pl.ds(qi * L, L)] = posm + trow
            ir[pl.ds(qi * L, L)] = posh + trow
            return c2

        lax.fori_loop(0, QB // L, _ranks, 0)

        c0 = pltpu.async_copy(pv_hbm.at[il], g0, sem)
        c1 = pltpu.async_copy(pv_hbm.at[ia], g1, sem)
        c2 = pltpu.async_copy(pv_hbm.at[ir], g2, sem)
        c3 = pltpu.async_copy(pzv_hbm.at[il], g3, sem)
        c4 = pltpu.async_copy(pzv_hbm.at[ia], g4, sem)
        c5 = pltpu.async_copy(pzv_hbm.at[ir], g5, sem)
        c0.wait()
        c1.wait()
        c2.wait()
        c3.wait()
        c4.wait()
        c5.wait()

        def _combine(qi, c2):
            zqv = plsc.load_gather(zq_s, [jnp.full((L,), bi * QB + qi,
                                                   jnp.int32)])
            wl = one - zqv * inv_bw
            wr = one + zqv * inv_bw
            for g in range(4):
                sl = pl.ds(g * L, L)
                pvl = g0[qi, sl]
                pva = g1[qi, sl]
                pvr = g2[qi, sl]
                pzl = g3[qi, sl]
                pza = g4[qi, sl]
                pzr = g5[qi, sl]
                left = (pva - pvl) * wl + (pza - pzl) * inv_bw
                right = (pvr - pva) * wr - (pzr - pza) * inv_bw
                outb[qi, sl] = (left + right) * scale
            return c2

        lax.fori_loop(0, QB, _combine, 0)
        pltpu.sync_copy(outb, ctx_hbm.at[pl.ds(base + bi * QB, QB)])
        return c

    lax.fori_loop(0, T // QB, _batch, 0)


def _sc_bump(zq, zk, v_rows, bw):
    mesh = plsc.VectorSubcoreMesh(core_axis_name="c", subcore_axis_name="s")
    f = pl.kernel(
        _sc_bump_body,
        mesh=mesh,
        compiler_params=pltpu.CompilerParams(
            needs_layout_passes=False, use_tc_tiling_on_sc=False),
        out_type=[
            jax.ShapeDtypeStruct((BH * T, D), jnp.float32),
            jax.ShapeDtypeStruct((BH * TROWS, D), jnp.float32),
            jax.ShapeDtypeStruct((BH * TROWS, D), jnp.float32),
        ],
        scratch_types=[
            pltpu.VMEM((T,), jnp.float32),
            pltpu.VMEM((T,), jnp.float32),
            pltpu.VMEM((T,), jnp.int32),
            pltpu.VMEM((CHUNK, D), jnp.float32),
            pltpu.VMEM((CHUNK, D), jnp.float32),
            pltpu.VMEM((CHUNK, D), jnp.float32),
            pltpu.VMEM((QB, D), jnp.float32),
            pltpu.VMEM((QB, D), jnp.float32),
            pltpu.VMEM((QB, D), jnp.float32),
            pltpu.VMEM((QB, D), jnp.float32),
            pltpu.VMEM((QB, D), jnp.float32),
            pltpu.VMEM((QB, D), jnp.float32),
            pltpu.VMEM((QB, D), jnp.float32),
            pltpu.VMEM((QB,), jnp.int32),
            pltpu.VMEM((QB,), jnp.int32),
            pltpu.VMEM((QB,), jnp.int32),
            pltpu.VMEM((BH * L,), jnp.float32),
            pltpu.SemaphoreType.DMA,
        ],
    )
    bw16 = jnp.broadcast_to(bw[:, None], (BH, L)).reshape(BH * L)
    ctx, _, _ = f(zq.reshape(BH * T), zk.reshape(BH * T), v_rows, bw16)
    return ctx


@jax.jit
def kernel(hidden_states, Wq, bq, Wk, bk, Wv, bv, Wproj, log_bandwidth):
    x = hidden_states.reshape(B * T, HID)
    w_qkv = jnp.concatenate([Wq.T, Wk.T, Wv.T], axis=1)
    b_qkv = jnp.concatenate([bq, bk, bv], axis=0)
    qkv = _matmul_bias(x, w_qkv, b_qkv, bm=512, bn=1024)
    q, k, v = jnp.split(qkv.reshape(B, T, 3 * HID), 3, axis=2)

    q_flat = q.reshape(B, T, H, D).transpose(0, 2, 1, 3).reshape(B * T, HID)
    k_flat = k.reshape(B, T, H, D).transpose(0, 2, 1, 3).reshape(B * T, HID)
    qk_flat = jnp.concatenate([q_flat, k_flat], axis=0)
    wp = jnp.zeros((HID, 128), jnp.float32).at[:, :H].set(Wproj.T)
    zqk = _matmul_bias(qk_flat, wp, jnp.zeros((128,), jnp.float32),
                       bm=1024, bn=128)[:, :H]
    zq = zqk[: B * T].reshape(B, T, H).transpose(0, 2, 1).reshape(BH, T)
    zk = zqk[B * T:].reshape(B, T, H).transpose(0, 2, 1).reshape(BH, T)

    v_rows = v.reshape(B, T, H, D).transpose(0, 2, 1, 3).reshape(BH * T, D)
    log_bw_bh = jnp.broadcast_to(log_bandwidth[None, :], (B, H)).reshape(BH)
    bw = _bandwidths(log_bw_bh).reshape(BH)

    ctx = _sc_bump(zq, zk, v_rows, bw)
    return ctx.reshape(B, H, T, D).transpose(0, 2, 1, 3).reshape(B, T, HID)


# speedup vs baseline: 7.9078x; 1.1603x over previous
import functools

import jax
import jax.numpy as jnp
from jax import lax
from jax.experimental import pallas as pl
from jax.experimental.pallas import tpu as pltpu
from jax.experimental.pallas import tpu_sc as plsc

B, T, HID, H = 2, 4096, 1024, 16
D = HID // H
BH = B * H
EPS = 1e-4
L = 16
NC, NS = 2, 16
CHUNK = 128
QB = 128
TROWS = T + 8


def _matmul_bias_kernel(x_ref, w_ref, b_ref, o_ref):
    o_ref[...] = (
        jnp.dot(x_ref[...], w_ref[...], preferred_element_type=jnp.float32)
        + b_ref[...]
    )


def _matmul_bias(x, w, bias, bm, bn):
    m, k = x.shape
    n = w.shape[1]
    return pl.pallas_call(
        _matmul_bias_kernel,
        grid=(m // bm, n // bn),
        in_specs=[
            pl.BlockSpec((bm, k), lambda i, j: (i, 0)),
            pl.BlockSpec((k, bn), lambda i, j: (0, j)),
            pl.BlockSpec((1, bn), lambda i, j: (0, j)),
        ],
        out_specs=pl.BlockSpec((bm, bn), lambda i, j: (i, j)),
        out_shape=jax.ShapeDtypeStruct((m, n), jnp.float32),
    )(x, w, bias.reshape(1, n))


def _bw_kernel(l_ref, o_ref):
    x = l_ref[...]
    sp = jnp.log1p(jnp.exp(-jnp.abs(x))) + jnp.maximum(x, 0.0) + EPS
    o_ref[...] = jnp.maximum(sp, EPS)


def _bandwidths(log_bw_bh):
    return pl.pallas_call(
        _bw_kernel,
        out_shape=jax.ShapeDtypeStruct((1, BH), jnp.float32),
    )(log_bw_bh.reshape(1, BH))


def _sc_bump_body(zq_hbm, zk_hbm, v_hbm, bw_hbm, ctx_hbm, pv_hbm, pzv_hbm,
                  zk_s, zq_s, pidx, va, pvb, zvb, g0, g1, g2, g3, g4, g5,
                  outb, il, ia, ir, bwv, sem):
    w = lax.axis_index("s") * NC + lax.axis_index("c")
    b = w // H
    h = w % H
    base = w * T
    trow = w * TROWS
    pltpu.sync_copy(zk_hbm.at[pl.ds(base, T)], zk_s)
    pltpu.sync_copy(zq_hbm.at[pl.ds(base, T)], zq_s)
    pltpu.sync_copy(bw_hbm, bwv)
    bw16 = bwv[pl.ds(w * L, L)]
    inv_bw = 1.0 / bw16
    lane = lax.iota(jnp.int32, L)

    vbase = b * T * H + h

    def _init(i, c):
        pidx[pl.ds(i * L, L)] = vbase + (i * L + lane) * H
        return c

    lax.fori_loop(0, T // L, _init, 0)

    def _vsort0(i, c):
        k, p = plsc.sort_key_val(zk_s[pl.ds(i * L, L)],
                                 pidx[pl.ds(i * L, L)])
        zk_s[pl.ds(i * L, L)] = k
        pidx[pl.ds(i * L, L)] = p
        return c

    lax.fori_loop(0, T // L, _vsort0, 0)

    npairs = T // (2 * L)
    for lvl in range(8):
        m = 32 << lvl
        half = max(m // (2 * L), 1)

        def _rev_merge(p, c, m=m, half=half):
            r = p // half
            i = p % half
            a0 = r * m + i * L
            b0 = r * m + m - L - i * L
            ka = zk_s[pl.ds(a0, L)]
            kb = lax.rev(zk_s[pl.ds(b0, L)], (0,))
            pa = pidx[pl.ds(a0, L)]
            pb = lax.rev(pidx[pl.ds(b0, L)], (0,))
            msk = ka <= kb
            zk_s[pl.ds(a0, L)] = jnp.where(msk, ka, kb)
            pidx[pl.ds(a0, L)] = jnp.where(msk, pa, pb)
            zk_s[pl.ds(b0, L)] = lax.rev(jnp.where(msk, kb, ka), (0,))
            pidx[pl.ds(b0, L)] = lax.rev(jnp.where(msk, pb, pa), (0,))
            return c

        lax.fori_loop(0, npairs, _rev_merge, 0)
        strd = m // 4
        while strd >= L:
            spb = strd // L

            def _cx(p, c, spb=spb, strd=strd):
                blk = p // spb
                off = p % spb
                a0 = (blk * 2 * spb + off) * L
                b0 = a0 + strd
                ka = zk_s[pl.ds(a0, L)]
                kb = zk_s[pl.ds(b0, L)]
                pa = pidx[pl.ds(a0, L)]
                pb = pidx[pl.ds(b0, L)]
                msk = ka <= kb
                zk_s[pl.ds(a0, L)] = jnp.where(msk, ka, kb)
                zk_s[pl.ds(b0, L)] = jnp.where(msk, kb, ka)
                pidx[pl.ds(a0, L)] = jnp.where(msk, pa, pb)
                pidx[pl.ds(b0, L)] = jnp.where(msk, pb, pa)
                return c

            lax.fori_loop(0, npairs, _cx, 0)
            strd //= 2

        def _vsort(i, c):
            k, p = plsc.sort_key_val(zk_s[pl.ds(i * L, L)],
                                     pidx[pl.ds(i * L, L)])
            zk_s[pl.ds(i * L, L)] = k
            pidx[pl.ds(i * L, L)] = p
            return c

        lax.fori_loop(0, T // L, _vsort, 0)

    zero = jnp.zeros((L,), jnp.float32)
    carry0 = (zero,) * 8
    for ci in range(T // CHUNK):
        pltpu.async_copy(v_hbm.at[pidx.at[pl.ds(ci * CHUNK, CHUNK)]], va,
                         sem).wait()

        def _row(i, cr, ci=ci):
            pv0, pv1, pv2, pv3, pz0, pz1, pz2, pz3 = cr
            zv = plsc.load_gather(zk_s, [jnp.full((L,), ci * CHUNK + i,
                                                  jnp.int32)])
            v0 = va[i, pl.ds(0, L)]
            v1 = va[i, pl.ds(L, L)]
            v2 = va[i, pl.ds(2 * L, L)]
            v3 = va[i, pl.ds(3 * L, L)]
            pvb[i, pl.ds(0, L)] = pv0
            pvb[i, pl.ds(L, L)] = pv1
            pvb[i, pl.ds(2 * L, L)] = pv2
            pvb[i, pl.ds(3 * L, L)] = pv3
            zvb[i, pl.ds(0, L)] = pz0
            zvb[i, pl.ds(L, L)] = pz1
            zvb[i, pl.ds(2 * L, L)] = pz2
            zvb[i, pl.ds(3 * L, L)] = pz3
            return (pv0 + v0, pv1 + v1, pv2 + v2, pv3 + v3,
                    pz0 + v0 * zv, pz1 + v1 * zv, pz2 + v2 * zv,
                    pz3 + v3 * zv)

        carry0 = lax.fori_loop(0, CHUNK, _row, carry0)
        pltpu.sync_copy(pvb, pv_hbm.at[pl.ds(trow + ci * CHUNK, CHUNK)])
        pltpu.sync_copy(zvb, pzv_hbm.at[pl.ds(trow + ci * CHUNK, CHUNK)])

    for g in range(4):
        pvb[0, pl.ds(g * L, L)] = carry0[g]
        zvb[0, pl.ds(g * L, L)] = carry0[4 + g]
    pltpu.sync_copy(pvb.at[0], pv_hbm.at[trow + T])
    pltpu.sync_copy(zvb.at[0], pzv_hbm.at[trow + T])

    one = jnp.float32(1.0)
    scale = jnp.float32(1.0 / T)

    def _batch(bi, c):
        def _ranks(qi, c2):
            q0 = bi * QB + qi * L
            zqv = zq_s[pl.ds(q0, L)]
            tl = zqv - bw16
            tm = zqv
            th = zqv + bw16
            posl = jnp.zeros((L,), jnp.int32)
            posm = jnp.zeros((L,), jnp.int32)
            posh = jnp.zeros((L,), jnp.int32)
            for k in range(12, -1, -1):
                bit = 1 << k
                for which in range(3):
                    pos = (posl, posm, posh)[which]
                    tgt = (tl, tm, th)[which]
                    npos = pos + bit
                    idx = jnp.minimum(npos, T) - 1
                    val = plsc.load_gather(zk_s, [idx])
                    pos = jnp.where((npos <= T) & (val < tgt), npos, pos)
                    if which == 0:
                        posl = pos
                    elif which == 1:
                        posm = pos
                    else:
                        posh = pos
            il[pl.ds(qi * L, L)] = posl + trow
            ia[pl.ds(qi * L, L)] = posm + trow
            ir[pl.ds(qi * L, L)] = posh + trow
            return c2

        lax.fori_loop(0, QB // L, _ranks, 0)

        c0 = pltpu.async_copy(pv_hbm.at[il], g0, sem)
        c1 = pltpu.async_copy(pv_hbm.at[ia], g1, sem)
        c2 = pltpu.async_copy(pv_hbm.at[ir], g2, sem)
        c3 = pltpu.async_copy(pzv_hbm.at[il], g3, sem)
        c4 = pltpu.async_copy(pzv_hbm.at[ia], g4, sem)
        c5 = pltpu.async_copy(pzv_hbm.at[ir], g5, sem)
        c0.wait()
        c1.wait()
        c2.wait()
        c3.wait()
        c4.wait()
        c5.wait()

        def _combine(qi, c2):
            zqv = plsc.load_gather(zq_s, [jnp.full((L,), bi * QB + qi,
                                                   jnp.int32)])
            wl = one - zqv * inv_bw
            wr = one + zqv * inv_bw
            for g in range(4):
                sl = pl.ds(g * L, L)
                pvl = g0[qi, sl]
                pva = g1[qi, sl]
                pvr = g2[qi, sl]
                pzl = g3[qi, sl]
                pza = g4[qi, sl]
                pzr = g5[qi, sl]
                left = (pva - pvl) * wl + (pza - pzl) * inv_bw
                right = (pvr - pva) * wr - (pzr - pza) * inv_bw
                outb[qi, sl] = (left + right) * scale
            return c2

        lax.fori_loop(0, QB, _combine, 0)
        pltpu.sync_copy(
            outb,
            ctx_hbm.at[pl.ds(b * T + bi * QB, QB), pl.ds(h * D, D)])
        return c

    lax.fori_loop(0, T // QB, _batch, 0)


def _sc_bump(zq, zk, v_rows, bw):
    mesh = plsc.VectorSubcoreMesh(core_axis_name="c", subcore_axis_name="s")
    f = pl.kernel(
        _sc_bump_body,
        mesh=mesh,
        compiler_params=pltpu.CompilerParams(
            needs_layout_passes=False, use_tc_tiling_on_sc=False),
        out_type=[
            jax.ShapeDtypeStruct((B * T, HID), jnp.float32),
            jax.ShapeDtypeStruct((BH * TROWS, D), jnp.float32),
            jax.ShapeDtypeStruct((BH * TROWS, D), jnp.float32),
        ],
        scratch_types=[
            pltpu.VMEM((T,), jnp.float32),
            pltpu.VMEM((T,), jnp.float32),
            pltpu.VMEM((T,), jnp.int32),
            pltpu.VMEM((CHUNK, D), jnp.float32),
            pltpu.VMEM((CHUNK, D), jnp.float32),
            pltpu.VMEM((CHUNK, D), jnp.float32),
            pltpu.VMEM((QB, D), jnp.float32),
            pltpu.VMEM((QB, D), jnp.float32),
            pltpu.VMEM((QB, D), jnp.float32),
            pltpu.VMEM((QB, D), jnp.float32),
            pltpu.VMEM((QB, D), jnp.float32),
            pltpu.VMEM((QB, D), jnp.float32),
            pltpu.VMEM((QB, D), jnp.float32),
            pltpu.VMEM((QB,), jnp.int32),
            pltpu.VMEM((QB,), jnp.int32),
            pltpu.VMEM((QB,), jnp.int32),
            pltpu.VMEM((BH * L,), jnp.float32),
            pltpu.SemaphoreType.DMA,
        ],
    )
    bw16 = jnp.broadcast_to(bw[:, None], (BH, L)).reshape(BH * L)
    ctx, _, _ = f(zq.reshape(BH * T), zk.reshape(BH * T), v_rows, bw16)
    return ctx


@jax.jit
def kernel(hidden_states, Wq, bq, Wk, bk, Wv, bv, Wproj, log_bandwidth):
    x = hidden_states.reshape(B * T, HID)
    w_qkv = jnp.concatenate([Wq.T, Wk.T, Wv.T], axis=1)
    b_qkv = jnp.concatenate([bq, bk, bv], axis=0)
    qkv = _matmul_bias(x, w_qkv, b_qkv, bm=512, bn=1024)
    q, k, v = jnp.split(qkv.reshape(B, T, 3 * HID), 3, axis=2)

    q_flat = q.reshape(B, T, H, D).transpose(0, 2, 1, 3).reshape(B * T, HID)
    k_flat = k.reshape(B, T, H, D).transpose(0, 2, 1, 3).reshape(B * T, HID)
    qk_flat = jnp.concatenate([q_flat, k_flat], axis=0)
    wp = jnp.zeros((HID, 128), jnp.float32).at[:, :H].set(Wproj.T)
    zqk = _matmul_bias(qk_flat, wp, jnp.zeros((128,), jnp.float32),
                       bm=1024, bn=128)[:, :H]
    zq = zqk[: B * T].reshape(B, T, H).transpose(0, 2, 1).reshape(BH, T)
    zk = zqk[B * T:].reshape(B, T, H).transpose(0, 2, 1).reshape(BH, T)

    v_rows = v.reshape(B * T * H, D)
    log_bw_bh = jnp.broadcast_to(log_bandwidth[None, :], (B, H)).reshape(BH)
    bw = _bandwidths(log_bw_bh).reshape(BH)

    ctx = _sc_bump(zq, zk, v_rows, bw)
    return ctx.reshape(B, T, HID)
